# baseline (speedup 1.0000x reference)
import functools

import jax
import jax.numpy as jnp
from jax import lax
from jax.experimental import pallas as pl
from jax.experimental.pallas import tpu as pltpu
from jax.experimental.pallas import tpu_sc as plsc

N_NODES = 10000
N_EDGES = 320000
D_FEAT = 128
HIDDEN = 16
NUM_CLASSES = 40

NC = 2
NS = 16
NW = NC * NS
EB = 128
N_PAD = 10240
ROWS_PER_TILE = N_PAD // NS

NCHUNKS = N_EDGES // EB
NBLK = NCHUNKS // NW
N_EXTRA = NCHUNKS - NBLK * NW


def _lane_broadcast(v16, t):
  dnums = lax.GatherDimensionNumbers(
      offset_dims=(), collapsed_slice_dims=(0,), start_index_map=(0,))
  idx = jnp.full((16, 1), t, jnp.int32)
  return lax.gather(v16, idx, dnums, (1,),
                    mode=lax.GatherScatterMode.PROMISE_IN_BOUNDS)


NBUF = 4

_SCRATCH = (
    [
        pltpu.VMEM((NBLK + 1, EB), jnp.int32),
        pltpu.VMEM((NBLK + 1, EB), jnp.int32),
        pltpu.VMEM(((NBLK + 1) * EB,), jnp.float32),
    ]
    + [pltpu.VMEM((EB, HIDDEN), jnp.float32)] * NBUF
    + [
        pltpu.VMEM((128, HIDDEN), jnp.float32),
        pltpu.VMEM_SHARED((N_PAD, HIDDEN), jnp.float32),
        pltpu.VMEM_SHARED((N_PAD, HIDDEN), jnp.float32),
    ]
    + [pltpu.SemaphoreType.DMA] * (2 * NBUF)
)


def _zero_acc(zero_v, acc_sh, s):
  for r in range(128):
    zero_v[r] = jnp.zeros((HIDDEN,), jnp.float32)
  for i in range(ROWS_PER_TILE // 128):
    pltpu.sync_copy(zero_v,
                    acc_sh.at[pl.ds(s * ROWS_PER_TILE + i * 128, 128)])


def _load_edges(ei_hbm, w_hbm, src_v, dst_v, w_v, wid, extra):
  c0 = wid * NBLK
  pltpu.sync_copy(ei_hbm.at[pl.ds(c0, NBLK), pl.ds(0, EB)],
                  src_v.at[pl.ds(0, NBLK)])
  pltpu.sync_copy(ei_hbm.at[pl.ds(c0, NBLK), pl.ds(EB, EB)],
                  dst_v.at[pl.ds(0, NBLK)])
  pltpu.sync_copy(w_hbm.at[pl.ds(c0 * EB, NBLK * EB)],
                  w_v.at[pl.ds(0, NBLK * EB)])

  @pl.when(extra)
  def _leftover():
    ce = NBLK * NW + wid
    pltpu.sync_copy(ei_hbm.at[ce, pl.ds(0, EB)], src_v.at[NBLK])
    pltpu.sync_copy(ei_hbm.at[ce, pl.ds(EB, EB)], dst_v.at[NBLK])
    pltpu.sync_copy(w_hbm.at[pl.ds(ce * EB, EB)],
                    w_v.at[pl.ds(NBLK * EB, EB)])


_MAIN_START = 2
_NITER = (NBLK - _MAIN_START - 2) // NBUF
_NEPI = NBLK - _MAIN_START - _NITER * NBUF
assert _MAIN_START + _NITER * NBUF + 1 < NBLK, "epilogue too short"


def _edge_loop(src_v, dst_v, w_v, rows, gsems, ssems, table_sh, acc_sh,
               extra):
  def gather(b, u):
    pltpu.async_copy(table_sh.at[src_v.at[b]], rows[u], gsems[u])

  def wait_gather(b, u):
    pltpu.make_async_copy(table_sh.at[src_v.at[b]], rows[u], gsems[u]).wait()

  def scale(b, u):
    rv = rows[u]
    wbase = b * EB
    for e in range(EB):
      w1 = w_v[pl.ds(wbase + e, 1)]
      rv[e] = rv[e] * w1[0]

  def scatter(b, u):
    pltpu.async_copy(rows[u], acc_sh.at[dst_v.at[b]], ssems[u], add=True)

  def wait_scatter(b, u):
    pltpu.make_async_copy(rows[u], acc_sh.at[dst_v.at[b]], ssems[u]).wait()

  for b in range(NBUF):
    gather(b, b % NBUF)
  for s in range(_MAIN_START):
    u = s % NBUF
    wait_gather(s, u)
    scale(s, u)
    scatter(s, u)

  def blk_body(i, carry):
    base = _MAIN_START + i * NBUF
    for k in range(NBUF):
      s = base + k
      u_cur = (_MAIN_START + k) % NBUF
      u_new = (_MAIN_START + k + 2) % NBUF
      wait_scatter(s - 2, u_new)
      gather(s + 2, u_new)
      wait_gather(s, u_cur)
      scale(s, u_cur)
      scatter(s, u_cur)
    return carry

  lax.fori_loop(0, _NITER, blk_body, 0)

  for s in range(NBLK - _NEPI, NBLK):
    u = s % NBUF
    wait_scatter(s - 2, (s - 2) % NBUF)
    if s + 2 < NBLK:
      gather(s + 2, (s + 2) % NBUF)
    wait_gather(s, u)
    scale(s, u)
    scatter(s, u)
  for s in range(NBLK - 2, NBLK):
    wait_scatter(s, s % NBUF)

  @pl.when(extra)
  def _leftover():
    gather(NBLK, 0)
    wait_gather(NBLK, 0)
    scale(NBLK, 0)
    scatter(NBLK, 0)
    wait_scatter(NBLK, 0)


def _write_partial(acc_sh, out_hbm, c, s):
  rbase = s * ROWS_PER_TILE
  acc_slice = acc_sh.at[pl.ds(rbase, ROWS_PER_TILE)]

  @pl.when(c == 0)
  def _lane0():
    pltpu.sync_copy(acc_slice,
                    out_hbm.at[pl.ds(rbase, ROWS_PER_TILE), pl.ds(0, HIDDEN)])

  @pl.when(c == 1)
  def _lane1():
    pltpu.sync_copy(
        acc_slice,
        out_hbm.at[pl.ds(rbase, ROWS_PER_TILE), pl.ds(HIDDEN, HIDDEN)])


_PARTIAL_TYPE = jax.ShapeDtypeStruct((N_PAD, 128), jnp.float32)
_N_LAST = N_NODES - (NS - 1) * ROWS_PER_TILE


def _make_agg1_kernel():
  mesh = plsc.VectorSubcoreMesh(core_axis_name="c", subcore_axis_name="s")

  @functools.partial(
      pl.kernel,
      mesh=mesh,
      out_type=_PARTIAL_TYPE,
      scratch_types=list(_SCRATCH),
      compiler_params=pltpu.CompilerParams(use_tc_tiling_on_sc=False),
  )
  def agg1(table_hbm, ei_hbm, w_hbm, out_hbm,
           src_v, dst_v, w_v, rows0_v, rows1_v, rows2_v, rows3_v, zero_v,
           table_sh, acc_sh, gs0, gs1, gs2, gs3, ss0, ss1, ss2, ss3):
    c = lax.axis_index("c")
    s = lax.axis_index("s")
    wid = c * NS + s
    extra = wid < N_EXTRA

    @pl.when(s < NS - 1)
    def _stage_full():
      pltpu.sync_copy(
          table_hbm.at[pl.ds(s * ROWS_PER_TILE, ROWS_PER_TILE),
                       pl.ds(0, HIDDEN)],
          table_sh.at[pl.ds(s * ROWS_PER_TILE, ROWS_PER_TILE)])

    @pl.when(s == NS - 1)
    def _stage_clip():
      pltpu.sync_copy(
          table_hbm.at[pl.ds((NS - 1) * ROWS_PER_TILE, _N_LAST),
                       pl.ds(0, HIDDEN)],
          table_sh.at[pl.ds((NS - 1) * ROWS_PER_TILE, _N_LAST)])

    _load_edges(ei_hbm, w_hbm, src_v, dst_v, w_v, wid, extra)
    _zero_acc(zero_v, acc_sh, s)
    plsc.subcore_barrier()
    _edge_loop(src_v, dst_v, w_v, (rows0_v, rows1_v, rows2_v, rows3_v),
               (gs0, gs1, gs2, gs3), (ss0, ss1, ss2, ss3),
               table_sh, acc_sh, extra)
    plsc.subcore_barrier()
    _write_partial(acc_sh, out_hbm, c, s)

  return agg1


def _make_agg2_kernel():
  mesh = plsc.VectorSubcoreMesh(core_axis_name="c", subcore_axis_name="s")

  @functools.partial(
      pl.kernel,
      mesh=mesh,
      out_type=_PARTIAL_TYPE,
      scratch_types=list(_SCRATCH) + [
          pltpu.VMEM((ROWS_PER_TILE, HIDDEN), jnp.float32),
          pltpu.VMEM((ROWS_PER_TILE, HIDDEN), jnp.float32),
          pltpu.VMEM((16,), jnp.float32),
      ],
      compiler_params=pltpu.CompilerParams(use_tc_tiling_on_sc=False),
  )
  def agg2(p_hbm, b0_hbm, ei_hbm, w_hbm, out_hbm,
           src_v, dst_v, w_v, rows0_v, rows1_v, rows2_v, rows3_v, zero_v,
           table_sh, acc_sh, gs0, gs1, gs2, gs3, ss0, ss1, ss2, ss3,
           pa_v, pb_v, b0_v):
    c = lax.axis_index("c")
    s = lax.axis_index("s")
    wid = c * NS + s
    extra = wid < N_EXTRA

    base = s * ROWS_PER_TILE
    pltpu.sync_copy(p_hbm.at[pl.ds(base, ROWS_PER_TILE), pl.ds(0, HIDDEN)],
                    pa_v)
    pltpu.sync_copy(
        p_hbm.at[pl.ds(base, ROWS_PER_TILE), pl.ds(HIDDEN, HIDDEN)],
        pb_v)
    pltpu.sync_copy(b0_hbm, b0_v)
    _load_edges(ei_hbm, w_hbm, src_v, dst_v, w_v, wid, extra)
    b0 = b0_v[...]

    def comb_body(i, carry):
      pa_v[i] = jnp.maximum(pa_v[i] + pb_v[i] + b0, 0.0)
      return carry

    lax.fori_loop(0, ROWS_PER_TILE, comb_body, 0)
    pltpu.sync_copy(pa_v, table_sh.at[pl.ds(base, ROWS_PER_TILE)])
    _zero_acc(zero_v, acc_sh, s)
    plsc.subcore_barrier()
    _edge_loop(src_v, dst_v, w_v, (rows0_v, rows1_v, rows2_v, rows3_v),
               (gs0, gs1, gs2, gs3), (ss0, ss1, ss2, ss3),
               table_sh, acc_sh, extra)
    plsc.subcore_barrier()
    _write_partial(acc_sh, out_hbm, c, s)

  return agg2


_agg1 = _make_agg1_kernel()
_agg2 = _make_agg2_kernel()


def _matmul0(x, W0pad):
  def body(x_ref, w_ref, o_ref):
    o_ref[...] = jnp.dot(x_ref[...], w_ref[...],
                         preferred_element_type=jnp.float32)

  return pl.pallas_call(
      body,
      out_shape=jax.ShapeDtypeStruct((N_NODES, 128), jnp.float32),
  )(x, W0pad)


def _final(q, W1, b1):
  def body(q_ref, w_ref, b_ref, o_ref):
    qv = q_ref[...]
    agg = (qv[:, :HIDDEN] + qv[:, HIDDEN:2 * HIDDEN])[:N_NODES]
    o_ref[...] = jnp.dot(w_ref[...].T, agg.T,
                         preferred_element_type=jnp.float32) + b_ref[...]

  return pl.pallas_call(
      body,
      out_shape=jax.ShapeDtypeStruct((NUM_CLASSES, N_NODES), jnp.float32),
  )(q, W1, b1.reshape(NUM_CLASSES, 1))


def kernel(x, updated_edge_index, normed_edge_weight, W0, b0, W1, b1):
  ei = updated_edge_index.astype(jnp.int32)
  ei3 = lax.reshape(ei, (2, NCHUNKS, EB))
  ei2 = lax.reshape(ei3, (NCHUNKS, 2 * EB), dimensions=(1, 0, 2))
  w = normed_edge_weight.astype(jnp.float32)

  W0pad = jnp.pad(W0, ((0, 0), (0, 128 - HIDDEN)))

  h1 = _matmul0(x, W0pad)
  p = _agg1(h1, ei2, w)
  q = _agg2(p, b0, ei2, w)
  return _final(q, W1, b1).T


# speedup vs baseline: 2.2120x; 2.2120x over previous
import functools

import jax
import jax.numpy as jnp
from jax import lax
from jax.experimental import pallas as pl
from jax.experimental.pallas import tpu as pltpu
from jax.experimental.pallas import tpu_sc as plsc

N_NODES = 10000
N_EDGES = 320000
D_FEAT = 128
HIDDEN = 16
NUM_CLASSES = 40

NC = 2
NS = 16
NW = NC * NS
EB = 128
N_PAD = 10240
ROWS_PER_TILE = N_PAD // NS

NCHUNKS = N_EDGES // EB
NBLK = NCHUNKS // NW
N_EXTRA = NCHUNKS - NBLK * NW


def _lane_broadcast(v16, t):
  dnums = lax.GatherDimensionNumbers(
      offset_dims=(), collapsed_slice_dims=(0,), start_index_map=(0,))
  idx = jnp.full((16, 1), t, jnp.int32)
  return lax.gather(v16, idx, dnums, (1,),
                    mode=lax.GatherScatterMode.PROMISE_IN_BOUNDS)


NBUF = 4

_SCRATCH = (
    [
        pltpu.VMEM((NBLK + 1, EB), jnp.int32),
        pltpu.VMEM((NBLK + 1, EB), jnp.int32),
        pltpu.VMEM(((NBLK + 1) * EB,), jnp.float32),
    ]
    + [pltpu.VMEM((EB, HIDDEN), jnp.float32)] * NBUF
    + [
        pltpu.VMEM((128, HIDDEN), jnp.float32),
        pltpu.VMEM_SHARED((N_PAD, HIDDEN), jnp.float32),
        pltpu.VMEM_SHARED((N_PAD, HIDDEN), jnp.float32),
    ]
    + [pltpu.SemaphoreType.DMA] * (2 * NBUF)
)


def _zero_acc(zero_v, acc_sh, s):
  for r in range(128):
    zero_v[r] = jnp.zeros((HIDDEN,), jnp.float32)
  for i in range(ROWS_PER_TILE // 128):
    pltpu.sync_copy(zero_v,
                    acc_sh.at[pl.ds(s * ROWS_PER_TILE + i * 128, 128)])


def _load_edges(ei_hbm, w_hbm, src_v, dst_v, w_v, wid, extra):
  c0 = wid * NBLK
  pltpu.sync_copy(ei_hbm.at[pl.ds(c0, NBLK), pl.ds(0, EB)],
                  src_v.at[pl.ds(0, NBLK)])
  pltpu.sync_copy(ei_hbm.at[pl.ds(c0, NBLK), pl.ds(EB, EB)],
                  dst_v.at[pl.ds(0, NBLK)])
  pltpu.sync_copy(w_hbm.at[pl.ds(c0 * EB, NBLK * EB)],
                  w_v.at[pl.ds(0, NBLK * EB)])

  @pl.when(extra)
  def _leftover():
    ce = NBLK * NW + wid
    pltpu.sync_copy(ei_hbm.at[ce, pl.ds(0, EB)], src_v.at[NBLK])
    pltpu.sync_copy(ei_hbm.at[ce, pl.ds(EB, EB)], dst_v.at[NBLK])
    pltpu.sync_copy(w_hbm.at[pl.ds(ce * EB, EB)],
                    w_v.at[pl.ds(NBLK * EB, EB)])


_MAIN_START = 2
_NITER = (NBLK - _MAIN_START - 2) // NBUF
_NEPI = NBLK - _MAIN_START - _NITER * NBUF
assert _MAIN_START + _NITER * NBUF + 1 < NBLK, "epilogue too short"


def _edge_loop(src_v, dst_v, w_v, rows, gsems, ssems, table_sh, acc_sh,
               extra):
  def gather(b, u):
    pltpu.async_copy(table_sh.at[src_v.at[b]], rows[u], gsems[u])

  def wait_gather(b, u):
    pltpu.make_async_copy(table_sh.at[src_v.at[b]], rows[u], gsems[u]).wait()

  def scale(b, u):
    rv = rows[u]
    wbase = b * EB
    for j in range(EB // 16):
      w16 = w_v[pl.ds(wbase + j * 16, 16)]
      for t in range(16):
        e = j * 16 + t
        rv[e] = rv[e] * _lane_broadcast(w16, t)

  def scatter(b, u):
    pltpu.async_copy(rows[u], acc_sh.at[dst_v.at[b]], ssems[u], add=True)

  def wait_scatter(b, u):
    pltpu.make_async_copy(rows[u], acc_sh.at[dst_v.at[b]], ssems[u]).wait()

  for b in range(NBUF):
    gather(b, b % NBUF)
  for s in range(_MAIN_START):
    u = s % NBUF
    wait_gather(s, u)
    scale(s, u)
    scatter(s, u)

  def blk_body(i, carry):
    base = _MAIN_START + i * NBUF
    for k in range(NBUF):
      s = base + k
      u_cur = (_MAIN_START + k) % NBUF
      u_new = (_MAIN_START + k + 2) % NBUF
      wait_scatter(s - 2, u_new)
      gather(s + 2, u_new)
      wait_gather(s, u_cur)
      scale(s, u_cur)
      scatter(s, u_cur)
    return carry

  lax.fori_loop(0, _NITER, blk_body, 0)

  for s in range(NBLK - _NEPI, NBLK):
    u = s % NBUF
    wait_scatter(s - 2, (s - 2) % NBUF)
    if s + 2 < NBLK:
      gather(s + 2, (s + 2) % NBUF)
    wait_gather(s, u)
    scale(s, u)
    scatter(s, u)
  for s in range(NBLK - 2, NBLK):
    wait_scatter(s, s % NBUF)

  @pl.when(extra)
  def _leftover():
    gather(NBLK, 0)
    wait_gather(NBLK, 0)
    scale(NBLK, 0)
    scatter(NBLK, 0)
    wait_scatter(NBLK, 0)


def _write_partial(acc_sh, out_hbm, c, s):
  rbase = s * ROWS_PER_TILE
  acc_slice = acc_sh.at[pl.ds(rbase, ROWS_PER_TILE)]

  @pl.when(c == 0)
  def _lane0():
    pltpu.sync_copy(acc_slice,
                    out_hbm.at[pl.ds(rbase, ROWS_PER_TILE), pl.ds(0, HIDDEN)])

  @pl.when(c == 1)
  def _lane1():
    pltpu.sync_copy(
        acc_slice,
        out_hbm.at[pl.ds(rbase, ROWS_PER_TILE), pl.ds(HIDDEN, HIDDEN)])


_PARTIAL_TYPE = jax.ShapeDtypeStruct((N_PAD, 128), jnp.float32)
_N_LAST = N_NODES - (NS - 1) * ROWS_PER_TILE


def _make_agg1_kernel():
  mesh = plsc.VectorSubcoreMesh(core_axis_name="c", subcore_axis_name="s")

  @functools.partial(
      pl.kernel,
      mesh=mesh,
      out_type=_PARTIAL_TYPE,
      scratch_types=list(_SCRATCH),
      compiler_params=pltpu.CompilerParams(use_tc_tiling_on_sc=False),
  )
  def agg1(table_hbm, ei_hbm, w_hbm, out_hbm,
           src_v, dst_v, w_v, rows0_v, rows1_v, rows2_v, rows3_v, zero_v,
           table_sh, acc_sh, gs0, gs1, gs2, gs3, ss0, ss1, ss2, ss3):
    c = lax.axis_index("c")
    s = lax.axis_index("s")
    wid = c * NS + s
    extra = wid < N_EXTRA

    @pl.when(s < NS - 1)
    def _stage_full():
      pltpu.sync_copy(
          table_hbm.at[pl.ds(s * ROWS_PER_TILE, ROWS_PER_TILE),
                       pl.ds(0, HIDDEN)],
          table_sh.at[pl.ds(s * ROWS_PER_TILE, ROWS_PER_TILE)])

    @pl.when(s == NS - 1)
    def _stage_clip():
      pltpu.sync_copy(
          table_hbm.at[pl.ds((NS - 1) * ROWS_PER_TILE, _N_LAST),
                       pl.ds(0, HIDDEN)],
          table_sh.at[pl.ds((NS - 1) * ROWS_PER_TILE, _N_LAST)])

    _load_edges(ei_hbm, w_hbm, src_v, dst_v, w_v, wid, extra)
    _zero_acc(zero_v, acc_sh, s)
    plsc.subcore_barrier()
    _edge_loop(src_v, dst_v, w_v, (rows0_v, rows1_v, rows2_v, rows3_v),
               (gs0, gs1, gs2, gs3), (ss0, ss1, ss2, ss3),
               table_sh, acc_sh, extra)
    plsc.subcore_barrier()
    _write_partial(acc_sh, out_hbm, c, s)

  return agg1


def _make_agg2_kernel():
  mesh = plsc.VectorSubcoreMesh(core_axis_name="c", subcore_axis_name="s")

  @functools.partial(
      pl.kernel,
      mesh=mesh,
      out_type=_PARTIAL_TYPE,
      scratch_types=list(_SCRATCH) + [
          pltpu.VMEM((ROWS_PER_TILE, HIDDEN), jnp.float32),
          pltpu.VMEM((ROWS_PER_TILE, HIDDEN), jnp.float32),
          pltpu.VMEM((16,), jnp.float32),
      ],
      compiler_params=pltpu.CompilerParams(use_tc_tiling_on_sc=False),
  )
  def agg2(p_hbm, b0_hbm, ei_hbm, w_hbm, out_hbm,
           src_v, dst_v, w_v, rows0_v, rows1_v, rows2_v, rows3_v, zero_v,
           table_sh, acc_sh, gs0, gs1, gs2, gs3, ss0, ss1, ss2, ss3,
           pa_v, pb_v, b0_v):
    c = lax.axis_index("c")
    s = lax.axis_index("s")
    wid = c * NS + s
    extra = wid < N_EXTRA

    base = s * ROWS_PER_TILE
    pltpu.sync_copy(p_hbm.at[pl.ds(base, ROWS_PER_TILE), pl.ds(0, HIDDEN)],
                    pa_v)
    pltpu.sync_copy(
        p_hbm.at[pl.ds(base, ROWS_PER_TILE), pl.ds(HIDDEN, HIDDEN)],
        pb_v)
    pltpu.sync_copy(b0_hbm, b0_v)
    _load_edges(ei_hbm, w_hbm, src_v, dst_v, w_v, wid, extra)
    b0 = b0_v[...]

    def comb_body(i, carry):
      pa_v[i] = jnp.maximum(pa_v[i] + pb_v[i] + b0, 0.0)
      return carry

    lax.fori_loop(0, ROWS_PER_TILE, comb_body, 0)
    pltpu.sync_copy(pa_v, table_sh.at[pl.ds(base, ROWS_PER_TILE)])
    _zero_acc(zero_v, acc_sh, s)
    plsc.subcore_barrier()
    _edge_loop(src_v, dst_v, w_v, (rows0_v, rows1_v, rows2_v, rows3_v),
               (gs0, gs1, gs2, gs3), (ss0, ss1, ss2, ss3),
               table_sh, acc_sh, extra)
    plsc.subcore_barrier()
    _write_partial(acc_sh, out_hbm, c, s)

  return agg2


_agg1 = _make_agg1_kernel()
_agg2 = _make_agg2_kernel()


def _matmul0(x, W0pad):
  def body(x_ref, w_ref, o_ref):
    o_ref[...] = jnp.dot(x_ref[...], w_ref[...],
                         preferred_element_type=jnp.float32)

  return pl.pallas_call(
      body,
      out_shape=jax.ShapeDtypeStruct((N_NODES, 128), jnp.float32),
  )(x, W0pad)


def _final(q, W1, b1):
  def body(q_ref, w_ref, b_ref, o_ref):
    qv = q_ref[...]
    agg = (qv[:, :HIDDEN] + qv[:, HIDDEN:2 * HIDDEN])[:N_NODES]
    o_ref[...] = jnp.dot(w_ref[...].T, agg.T,
                         preferred_element_type=jnp.float32) + b_ref[...]

  return pl.pallas_call(
      body,
      out_shape=jax.ShapeDtypeStruct((NUM_CLASSES, N_NODES), jnp.float32),
  )(q, W1, b1.reshape(NUM_CLASSES, 1))


def kernel(x, updated_edge_index, normed_edge_weight, W0, b0, W1, b1):
  ei = updated_edge_index.astype(jnp.int32)
  ei3 = lax.reshape(ei, (2, NCHUNKS, EB))
  ei2 = lax.reshape(ei3, (NCHUNKS, 2 * EB), dimensions=(1, 0, 2))
  w = normed_edge_weight.astype(jnp.float32)

  W0pad = jnp.pad(W0, ((0, 0), (0, 128 - HIDDEN)))

  h1 = _matmul0(x, W0pad)
  p = _agg1(h1, ei2, w)
  q = _agg2(p, b0, ei2, w)
  return _final(q, W1, b1).T


# speedup vs baseline: 2.4641x; 1.1139x over previous
import functools

import jax
import jax.numpy as jnp
from jax import lax
from jax.experimental import pallas as pl
from jax.experimental.pallas import tpu as pltpu
from jax.experimental.pallas import tpu_sc as plsc

N_NODES = 10000
N_EDGES = 320000
D_FEAT = 128
HIDDEN = 16
NUM_CLASSES = 40

NC = 2
NS = 16
NW = NC * NS
EB = 128
N_PAD = 10240
ROWS_PER_TILE = N_PAD // NS

NCHUNKS = N_EDGES // EB
NBLK = NCHUNKS // NW
N_EXTRA = NCHUNKS - NBLK * NW


def _lane_broadcast(v16, t):
  dnums = lax.GatherDimensionNumbers(
      offset_dims=(), collapsed_slice_dims=(0,), start_index_map=(0,))
  idx = jnp.full((16, 1), t, jnp.int32)
  return lax.gather(v16, idx, dnums, (1,),
                    mode=lax.GatherScatterMode.PROMISE_IN_BOUNDS)


NBUF = 4

_SCRATCH = (
    [
        pltpu.VMEM((2 * (NBLK + 1), EB), jnp.int32),
        pltpu.VMEM(((NBLK + 1) * EB,), jnp.float32),
    ]
    + [pltpu.VMEM((EB, HIDDEN), jnp.float32)] * NBUF
    + [
        pltpu.VMEM((128, HIDDEN), jnp.float32),
        pltpu.VMEM_SHARED((N_PAD, HIDDEN), jnp.float32),
        pltpu.VMEM_SHARED((N_PAD, HIDDEN), jnp.float32),
    ]
    + [pltpu.SemaphoreType.DMA] * (2 * NBUF)
)


def _zero_acc(zero_v, acc_sh, s):
  for r in range(128):
    zero_v[r] = jnp.zeros((HIDDEN,), jnp.float32)
  for i in range(ROWS_PER_TILE // 128):
    pltpu.sync_copy(zero_v,
                    acc_sh.at[pl.ds(s * ROWS_PER_TILE + i * 128, 128)])


def _load_edges(ei_hbm, w_hbm, sd_v, w_v, wid, extra):
  c0 = wid * NBLK
  pltpu.sync_copy(ei_hbm.at[pl.ds(2 * c0, 2 * NBLK)],
                  sd_v.at[pl.ds(0, 2 * NBLK)])
  pltpu.sync_copy(w_hbm.at[pl.ds(c0 * EB, NBLK * EB)],
                  w_v.at[pl.ds(0, NBLK * EB)])

  @pl.when(extra)
  def _leftover():
    ce = NBLK * NW + wid
    pltpu.sync_copy(ei_hbm.at[pl.ds(2 * ce, 2)], sd_v.at[pl.ds(2 * NBLK, 2)])
    pltpu.sync_copy(w_hbm.at[pl.ds(ce * EB, EB)],
                    w_v.at[pl.ds(NBLK * EB, EB)])


_MAIN_START = 2
_NITER = (NBLK - _MAIN_START - 2) // NBUF
_NEPI = NBLK - _MAIN_START - _NITER * NBUF
assert _MAIN_START + _NITER * NBUF + 1 < NBLK, "epilogue too short"


def _edge_loop(sd_v, w_v, rows, gsems, ssems, table_sh, acc_sh, extra):
  def gather(b, u):
    pltpu.async_copy(table_sh.at[sd_v.at[2 * b]], rows[u], gsems[u])

  def wait_gather(b, u):
    pltpu.make_async_copy(table_sh.at[sd_v.at[2 * b]], rows[u],
                          gsems[u]).wait()

  def scale(b, u):
    rv = rows[u]
    wbase = b * EB
    for j in range(EB // 16):
      w16 = w_v[pl.ds(wbase + j * 16, 16)]
      for t in range(16):
        e = j * 16 + t
        rv[e] = rv[e] * _lane_broadcast(w16, t)

  def scatter(b, u):
    pltpu.async_copy(rows[u], acc_sh.at[sd_v.at[2 * b + 1]], ssems[u],
                     add=True)

  def wait_scatter(b, u):
    pltpu.make_async_copy(rows[u], acc_sh.at[sd_v.at[2 * b + 1]],
                          ssems[u]).wait()

  for b in range(NBUF):
    gather(b, b % NBUF)
  for s in range(_MAIN_START):
    u = s % NBUF
    wait_gather(s, u)
    scale(s, u)
    scatter(s, u)

  def blk_body(i, carry):
    base = _MAIN_START + i * NBUF
    for k in range(NBUF):
      s = base + k
      u_cur = (_MAIN_START + k) % NBUF
      u_new = (_MAIN_START + k + 2) % NBUF
      wait_scatter(s - 2, u_new)
      gather(s + 2, u_new)
      wait_gather(s, u_cur)
      scale(s, u_cur)
      scatter(s, u_cur)
    return carry

  lax.fori_loop(0, _NITER, blk_body, 0)

  for s in range(NBLK - _NEPI, NBLK):
    u = s % NBUF
    wait_scatter(s - 2, (s - 2) % NBUF)
    if s + 2 < NBLK:
      gather(s + 2, (s + 2) % NBUF)
    wait_gather(s, u)
    scale(s, u)
    scatter(s, u)
  for s in range(NBLK - 2, NBLK):
    wait_scatter(s, s % NBUF)

  @pl.when(extra)
  def _leftover():
    gather(NBLK, 0)
    wait_gather(NBLK, 0)
    scale(NBLK, 0)
    scatter(NBLK, 0)
    wait_scatter(NBLK, 0)


def _write_partial(acc_sh, out_hbm, c, s):
  rbase = s * ROWS_PER_TILE
  acc_slice = acc_sh.at[pl.ds(rbase, ROWS_PER_TILE)]

  @pl.when(c == 0)
  def _lane0():
    pltpu.sync_copy(acc_slice,
                    out_hbm.at[pl.ds(rbase, ROWS_PER_TILE), pl.ds(0, HIDDEN)])

  @pl.when(c == 1)
  def _lane1():
    pltpu.sync_copy(
        acc_slice,
        out_hbm.at[pl.ds(rbase, ROWS_PER_TILE), pl.ds(HIDDEN, HIDDEN)])


_PARTIAL_TYPE = jax.ShapeDtypeStruct((N_PAD, 128), jnp.float32)
_N_LAST = N_NODES - (NS - 1) * ROWS_PER_TILE


def _make_agg1_kernel():
  mesh = plsc.VectorSubcoreMesh(core_axis_name="c", subcore_axis_name="s")

  @functools.partial(
      pl.kernel,
      mesh=mesh,
      out_type=_PARTIAL_TYPE,
      scratch_types=list(_SCRATCH),
      compiler_params=pltpu.CompilerParams(use_tc_tiling_on_sc=False),
  )
  def agg1(table_hbm, ei_hbm, w_hbm, out_hbm,
           sd_v, w_v, rows0_v, rows1_v, rows2_v, rows3_v, zero_v,
           table_sh, acc_sh, gs0, gs1, gs2, gs3, ss0, ss1, ss2, ss3):
    c = lax.axis_index("c")
    s = lax.axis_index("s")
    wid = c * NS + s
    extra = wid < N_EXTRA

    @pl.when(s < NS - 1)
    def _stage_full():
      pltpu.sync_copy(
          table_hbm.at[pl.ds(s * ROWS_PER_TILE, ROWS_PER_TILE),
                       pl.ds(0, HIDDEN)],
          table_sh.at[pl.ds(s * ROWS_PER_TILE, ROWS_PER_TILE)])

    @pl.when(s == NS - 1)
    def _stage_clip():
      pltpu.sync_copy(
          table_hbm.at[pl.ds((NS - 1) * ROWS_PER_TILE, _N_LAST),
                       pl.ds(0, HIDDEN)],
          table_sh.at[pl.ds((NS - 1) * ROWS_PER_TILE, _N_LAST)])

    _load_edges(ei_hbm, w_hbm, sd_v, w_v, wid, extra)
    _zero_acc(zero_v, acc_sh, s)
    plsc.subcore_barrier()
    _edge_loop(sd_v, w_v, (rows0_v, rows1_v, rows2_v, rows3_v),
               (gs0, gs1, gs2, gs3), (ss0, ss1, ss2, ss3),
               table_sh, acc_sh, extra)
    plsc.subcore_barrier()
    _write_partial(acc_sh, out_hbm, c, s)

  return agg1


def _make_agg2_kernel():
  mesh = plsc.VectorSubcoreMesh(core_axis_name="c", subcore_axis_name="s")

  @functools.partial(
      pl.kernel,
      mesh=mesh,
      out_type=_PARTIAL_TYPE,
      scratch_types=list(_SCRATCH) + [
          pltpu.VMEM((ROWS_PER_TILE, HIDDEN), jnp.float32),
          pltpu.VMEM((ROWS_PER_TILE, HIDDEN), jnp.float32),
          pltpu.VMEM((16,), jnp.float32),
      ],
      compiler_params=pltpu.CompilerParams(use_tc_tiling_on_sc=False),
  )
  def agg2(p_hbm, b0_hbm, ei_hbm, w_hbm, out_hbm,
           sd_v, w_v, rows0_v, rows1_v, rows2_v, rows3_v, zero_v,
           table_sh, acc_sh, gs0, gs1, gs2, gs3, ss0, ss1, ss2, ss3,
           pa_v, pb_v, b0_v):
    c = lax.axis_index("c")
    s = lax.axis_index("s")
    wid = c * NS + s
    extra = wid < N_EXTRA

    base = s * ROWS_PER_TILE
    pltpu.sync_copy(p_hbm.at[pl.ds(base, ROWS_PER_TILE), pl.ds(0, HIDDEN)],
                    pa_v)
    pltpu.sync_copy(
        p_hbm.at[pl.ds(base, ROWS_PER_TILE), pl.ds(HIDDEN, HIDDEN)],
        pb_v)
    pltpu.sync_copy(b0_hbm, b0_v)
    _load_edges(ei_hbm, w_hbm, sd_v, w_v, wid, extra)
    b0 = b0_v[...]

    def comb_body(i, carry):
      pa_v[i] = jnp.maximum(pa_v[i] + pb_v[i] + b0, 0.0)
      return carry

    lax.fori_loop(0, ROWS_PER_TILE, comb_body, 0)
    pltpu.sync_copy(pa_v, table_sh.at[pl.ds(base, ROWS_PER_TILE)])
    _zero_acc(zero_v, acc_sh, s)
    plsc.subcore_barrier()
    _edge_loop(sd_v, w_v, (rows0_v, rows1_v, rows2_v, rows3_v),
               (gs0, gs1, gs2, gs3), (ss0, ss1, ss2, ss3),
               table_sh, acc_sh, extra)
    plsc.subcore_barrier()
    _write_partial(acc_sh, out_hbm, c, s)

  return agg2


_agg1 = _make_agg1_kernel()
_agg2 = _make_agg2_kernel()


def _matmul0(x, W0pad, ei):
  def body(x_ref, w_ref, ei_ref, o_ref, o2_ref):
    o_ref[...] = jnp.dot(x_ref[...], w_ref[...],
                         preferred_element_type=jnp.float32)
    e2 = ei_ref[...].reshape(2, NCHUNKS, EB)
    o2_ref[...] = jnp.swapaxes(e2, 0, 1).reshape(2 * NCHUNKS, EB)

  return pl.pallas_call(
      body,
      out_shape=(jax.ShapeDtypeStruct((N_NODES, 128), jnp.float32),
                 jax.ShapeDtypeStruct((2 * NCHUNKS, EB), jnp.int32)),
  )(x, W0pad, ei)


def _final(q, W1, b1):
  def body(q_ref, w_ref, b_ref, o_ref):
    qv = q_ref[...]
    agg = (qv[:, :HIDDEN] + qv[:, HIDDEN:2 * HIDDEN])[:N_NODES]
    o_ref[...] = jnp.dot(w_ref[...].T, agg.T,
                         preferred_element_type=jnp.float32) + b_ref[...]

  return pl.pallas_call(
      body,
      out_shape=jax.ShapeDtypeStruct((NUM_CLASSES, N_NODES), jnp.float32),
  )(q, W1, b1.reshape(NUM_CLASSES, 1))


def kernel(x, updated_edge_index, normed_edge_weight, W0, b0, W1, b1):
  ei = updated_edge_index.astype(jnp.int32)
  w = normed_edge_weight.astype(jnp.float32)

  W0pad = jnp.pad(W0, ((0, 0), (0, 128 - HIDDEN)))

  h1, ei2 = _matmul0(x, W0pad, ei)
  p = _agg1(h1, ei2, w)
  q = _agg2(p, b0, ei2, w)
  return _final(q, W1, b1).T
